# baseline (speedup 1.0000x reference)
import functools

import jax
import jax.numpy as jnp
from jax import lax
from jax.experimental import pallas as pl
from jax.experimental.pallas import tpu as pltpu
from jax.experimental.pallas import tpu_sc as plsc


def _sc_broadcast(pe_table, nb, S, D):
    info = plsc.get_sparse_core_info()
    NW = info.num_cores * info.num_subcores
    rows_per_w = S // NW
    CH = min(32, rows_per_w)
    nchunk = rows_per_w // CH

    mesh = plsc.VectorSubcoreMesh(core_axis_name="c", subcore_axis_name="s")

    @functools.partial(
        pl.kernel,
        out_type=jax.ShapeDtypeStruct((nb, S, D), pe_table.dtype),
        mesh=mesh,
        scratch_types=[
            pltpu.VMEM((CH, D), pe_table.dtype),
            pltpu.VMEM((CH, D), pe_table.dtype),
            pltpu.SemaphoreType.DMA,
            pltpu.SemaphoreType.DMA,
            pltpu.SemaphoreType.DMA,
            pltpu.SemaphoreType.DMA,
        ],
    )
    def sc_copy(pe_hbm, out_hbm, buf0, buf1, isem0, isem1, osem0, osem1):
        wid = lax.axis_index("s") * info.num_cores + lax.axis_index("c")
        base = wid * rows_per_w
        bufs = (buf0, buf1)
        isems = (isem0, isem1)
        osems = (osem0, osem1)

        def in_cp(ci):
            return pltpu.make_async_copy(
                pe_hbm.at[pl.ds(base + ci * CH, CH)], bufs[ci % 2], isems[ci % 2]
            )

        def out_cp(ci, b):
            return pltpu.make_async_copy(
                bufs[ci % 2], out_hbm.at[b, pl.ds(base + ci * CH, CH)], osems[ci % 2]
            )

        in_cp(0).start()
        for ci in range(nchunk):
            in_cp(ci).wait()
            if ci >= 1:
                for b in range(nb):
                    out_cp(ci - 1, b).wait()
            if ci + 1 < nchunk:
                in_cp(ci + 1).start()
            for b in range(nb):
                out_cp(ci, b).start()
        for b in range(nb):
            out_cp(nchunk - 1, b).wait()

    return sc_copy(pe_table)


def _tc_broadcast(pe_table, nb, S, D):
    NCHUNK = 8
    CS = S // NCHUNK

    def body(pe_hbm, o_hbm, vmem, in_sems, out_sems):
        def in_copy(c):
            return pltpu.make_async_copy(
                pe_hbm.at[pl.ds(c * CS, CS)],
                vmem.at[pl.ds(c * CS, CS)],
                in_sems.at[c],
            )

        def out_copy(c, b):
            return pltpu.make_async_copy(
                vmem.at[pl.ds(c * CS, CS)],
                o_hbm.at[b, pl.ds(c * CS, CS)],
                out_sems.at[c, b],
            )

        for c in range(NCHUNK):
            in_copy(c).start()
        for c in range(NCHUNK):
            in_copy(c).wait()
            for b in range(nb):
                out_copy(c, b).start()
        for c in range(NCHUNK):
            for b in range(nb):
                out_copy(c, b).wait()

    return pl.pallas_call(
        body,
        in_specs=[pl.BlockSpec(memory_space=pl.ANY)],
        out_specs=pl.BlockSpec(memory_space=pl.ANY),
        out_shape=jax.ShapeDtypeStruct((nb, S, D), pe_table.dtype),
        scratch_shapes=[
            pltpu.VMEM((S, D), pe_table.dtype),
            pltpu.SemaphoreType.DMA((NCHUNK,)),
            pltpu.SemaphoreType.DMA((NCHUNK, nb)),
        ],
    )(pe_table)


def kernel(x, pe_table):
    B, S, D = x.shape
    NB_SC = B // 2
    pe = pe_table[:S]
    sc_out = _sc_broadcast(pe, NB_SC, S, D)
    tc_out = _tc_broadcast(pe, B - NB_SC, S, D)
    return jnp.concatenate([tc_out, sc_out], axis=0)


# speedup vs baseline: 2.3918x; 2.3918x over previous
import functools

import jax
import jax.numpy as jnp
from jax import lax
from jax.experimental import pallas as pl
from jax.experimental.pallas import tpu as pltpu
from jax.experimental.pallas import tpu_sc as plsc


def _sc_broadcast(pe_table, B, S, D, split):
    info = plsc.get_sparse_core_info()
    NW = info.num_cores * info.num_subcores
    rows = S - split
    rows_per_w = rows // NW
    CH = min(32, rows_per_w)
    nchunk = rows_per_w // CH

    mesh = plsc.VectorSubcoreMesh(core_axis_name="c", subcore_axis_name="s")

    @functools.partial(
        pl.kernel,
        out_type=jax.ShapeDtypeStruct((B, S, D), pe_table.dtype),
        mesh=mesh,
        scratch_types=[
            pltpu.VMEM((CH, D), pe_table.dtype),
            pltpu.VMEM((CH, D), pe_table.dtype),
            pltpu.SemaphoreType.DMA,
            pltpu.SemaphoreType.DMA,
            pltpu.SemaphoreType.DMA,
            pltpu.SemaphoreType.DMA,
        ],
    )
    def sc_copy(pe_hbm, out_hbm, buf0, buf1, isem0, isem1, osem0, osem1):
        wid = lax.axis_index("s") * info.num_cores + lax.axis_index("c")
        base = split + wid * rows_per_w
        bufs = (buf0, buf1)
        isems = (isem0, isem1)
        osems = (osem0, osem1)

        def in_cp(ci):
            return pltpu.make_async_copy(
                pe_hbm.at[pl.ds(base + ci * CH, CH)], bufs[ci % 2], isems[ci % 2]
            )

        def out_cp(ci, b):
            return pltpu.make_async_copy(
                bufs[ci % 2], out_hbm.at[b, pl.ds(base + ci * CH, CH)], osems[ci % 2]
            )

        in_cp(0).start()
        for ci in range(nchunk):
            in_cp(ci).wait()
            if ci >= 1:
                for b in range(B):
                    out_cp(ci - 1, b).wait()
            if ci + 1 < nchunk:
                in_cp(ci + 1).start()
            for b in range(B):
                out_cp(ci, b).start()
        for b in range(B):
            out_cp(nchunk - 1, b).wait()

    return sc_copy(pe_table)


def _tc_broadcast(pe_table, prev, B, S, D, split):
    NCHUNK = 8
    CS = split // NCHUNK

    def body(pe_hbm, prev_hbm, o_hbm, vmem, in_sems, out_sems):
        def in_copy(c):
            return pltpu.make_async_copy(
                pe_hbm.at[pl.ds(c * CS, CS)],
                vmem.at[pl.ds(c * CS, CS)],
                in_sems.at[c],
            )

        def out_copy(c, b):
            return pltpu.make_async_copy(
                vmem.at[pl.ds(c * CS, CS)],
                o_hbm.at[b, pl.ds(c * CS, CS)],
                out_sems.at[c, b],
            )

        for c in range(NCHUNK):
            in_copy(c).start()
        for c in range(NCHUNK):
            in_copy(c).wait()
            for b in range(B):
                out_copy(c, b).start()
        for c in range(NCHUNK):
            for b in range(B):
                out_copy(c, b).wait()

    return pl.pallas_call(
        body,
        in_specs=[
            pl.BlockSpec(memory_space=pl.ANY),
            pl.BlockSpec(memory_space=pl.ANY),
        ],
        out_specs=pl.BlockSpec(memory_space=pl.ANY),
        out_shape=jax.ShapeDtypeStruct((B, S, D), pe_table.dtype),
        input_output_aliases={1: 0},
        scratch_shapes=[
            pltpu.VMEM((split, D), pe_table.dtype),
            pltpu.SemaphoreType.DMA((NCHUNK,)),
            pltpu.SemaphoreType.DMA((NCHUNK, B)),
        ],
    )(pe_table, prev)


def kernel(x, pe_table):
    B, S, D = x.shape
    SPLIT = 7 * S // 8
    pe = pe_table[:S]
    partial_out = _sc_broadcast(pe, B, S, D, SPLIT)
    return _tc_broadcast(pe, partial_out, B, S, D, SPLIT)
